# baseline (speedup 1.0000x reference)
import functools

import jax
import jax.numpy as jnp
from jax import lax
from jax.experimental import pallas as pl
from jax.experimental.pallas import tpu as pltpu

_BN_EPS = 1e-5


def _pack_conv_w(w_oihw, cinp):
    cout, cin, k, _ = w_oihw.shape
    kp = k + 1
    w = jnp.pad(w_oihw, ((0, 0), (0, cinp - cin), (0, 0), (0, 0)))
    wt = jnp.transpose(w, (2, 3, 0, 1))
    big = jnp.zeros((4, kp, kp, cout, cinp), jnp.float32)
    for dh in range(2):
        for dw in range(2):
            big = big.at[2 * dh + dw, dh:dh + k, dw:dw + k].set(wt)
    return jnp.transpose(big, (0, 3, 2, 1, 4)).reshape(4 * cout, kp * kp * cinp)


def _conv_pool_bn_kernel(x_ref, w_ref, bss_ref, o_ref, *, K, Cinp, Cout, R, P, Bp):
    j = pl.program_id(1)
    kp = K + 1
    bias = bss_ref[0:Cout]
    scale = bss_ref[Cout:2 * Cout]
    shift = bss_ref[2 * Cout:3 * Cout]
    w = w_ref[...]
    for r0 in range(0, R, P):
        taps = []
        for t in range(kp):
            q, c = t % 2, t // 2
            cols = [
                x_ref[0, q,
                      pl.ds(pl.multiple_of(2 * (j * R + r0 + p) * Cinp, 2 * Cinp),
                            kp * Cinp),
                      pl.ds(c, Bp)]
                for p in range(P)
            ]
            taps.append(cols[0] if P == 1 else jnp.concatenate(cols, axis=1))
        slab = taps[0] if kp == 1 else jnp.concatenate(taps, axis=0)
        acc = jnp.dot(w, slab.astype(jnp.bfloat16),
                      preferred_element_type=jnp.float32)
        pooled = jnp.maximum(jnp.maximum(acc[0:Cout], acc[Cout:2 * Cout]),
                             jnp.maximum(acc[2 * Cout:3 * Cout], acc[3 * Cout:]))
        y = jnp.maximum(pooled + bias, 0.0) * scale + shift
        for p in range(P):
            o_ref[0, r0 + p] = y[:, p * Bp:(p + 1) * Bp].astype(o_ref.dtype)


def _conv_stage(x_nhcw, w_oihw, b, gamma, beta, mean, var, *, pad, P, R):
    n, h, cin, w = x_nhcw.shape
    cout, cin_w, k, _ = w_oihw.shape
    cinp = ((cin + 7) // 8) * 8
    hp, wp = h + 2 * pad, w + 2 * pad
    a, bp = (hp - k + 1) // 2, (wp - k + 1) // 2
    kp, wq = k + 1, wp // 2

    xp = jnp.pad(x_nhcw, ((0, 0), (pad, pad), (0, cinp - cin), (pad, pad)))
    xq = jnp.stack([xp[:, :, :, 0::2], xp[:, :, :, 1::2]], axis=1)
    xq = xq.reshape(n, 2, hp * cinp, wq)

    wpk = _pack_conv_w(w_oihw, cinp).astype(jnp.bfloat16)
    scale = gamma * lax.rsqrt(var + _BN_EPS)
    shift = beta - mean * scale
    bss = jnp.concatenate([b, scale, shift]).astype(jnp.float32).reshape(3 * cout, 1)

    kern = functools.partial(_conv_pool_bn_kernel, K=k, Cinp=cinp, Cout=cout,
                             R=R, P=P, Bp=bp)
    return pl.pallas_call(
        kern,
        out_shape=jax.ShapeDtypeStruct((n, a, cout, bp), jnp.float32),
        grid_spec=pltpu.PrefetchScalarGridSpec(
            num_scalar_prefetch=0,
            grid=(n, a // R),
            in_specs=[
                pl.BlockSpec((1, 2, hp * cinp, wq), lambda ni, ji: (ni, 0, 0, 0)),
                pl.BlockSpec((4 * cout, kp * kp * cinp), lambda ni, ji: (0, 0)),
                pl.BlockSpec((3 * cout, 1), lambda ni, ji: (0, 0)),
            ],
            out_specs=pl.BlockSpec((1, R, cout, bp), lambda ni, ji: (ni, ji, 0, 0)),
        ),
        compiler_params=pltpu.CompilerParams(
            dimension_semantics=("parallel", "parallel")),
    )(xq, wpk, bss)


def _mlp_kernel(x4_ref, w4_ref, b0_ref, w1_ref, b1_ref, o_ref, *, N, M0):
    y = jnp.dot(x4_ref[...], w4_ref[...], preferred_element_type=jnp.float32)
    h = (y[0:N, 0:M0] + y[N:2 * N, M0:2 * M0] + y[2 * N:3 * N, 2 * M0:3 * M0]
         + y[3 * N:4 * N, 3 * M0:4 * M0] + b0_ref[...])
    h = jnp.maximum(h, 0.0).astype(jnp.bfloat16)
    o_ref[...] = jnp.dot(h, w1_ref[...],
                         preferred_element_type=jnp.float32) + b1_ref[...]


def _mlp(o_nhcw, lw0, lb0, lw1, lb1):
    n, hh, cc, ww = o_nhcw.shape
    kdim, m0 = lw0.shape
    m1 = lw1.shape[1]
    kc = kdim // 4
    lw0p = lw0.reshape(cc, hh, ww, m0).transpose(1, 0, 2, 3).reshape(kdim, m0)
    x = o_nhcw.reshape(n, kdim)
    x4 = jnp.concatenate([x[:, i * kc:(i + 1) * kc] for i in range(4)],
                         axis=0).astype(jnp.bfloat16)
    w4 = jnp.concatenate([lw0p[i * kc:(i + 1) * kc] for i in range(4)],
                         axis=1).astype(jnp.bfloat16)
    return pl.pallas_call(
        functools.partial(_mlp_kernel, N=n, M0=m0),
        out_shape=jax.ShapeDtypeStruct((n, m1), jnp.float32),
        grid_spec=pltpu.PrefetchScalarGridSpec(
            num_scalar_prefetch=0,
            grid=(1,),
            in_specs=[
                pl.BlockSpec((4 * n, kc), lambda i: (0, 0)),
                pl.BlockSpec((kc, 4 * m0), lambda i: (0, 0)),
                pl.BlockSpec((1, m0), lambda i: (0, 0)),
                pl.BlockSpec((m0, m1), lambda i: (0, 0)),
                pl.BlockSpec((1, m1), lambda i: (0, 0)),
            ],
            out_specs=pl.BlockSpec((n, m1), lambda i: (0, 0)),
        ),
        compiler_params=pltpu.CompilerParams(
            dimension_semantics=("arbitrary",),
            vmem_limit_bytes=64 * 1024 * 1024),
    )(x4, w4, lb0.astype(jnp.float32).reshape(1, m0),
      lw1.astype(jnp.bfloat16), lb1.astype(jnp.float32).reshape(1, m1))


def kernel(x, w0, b0, w1, b1, w2, b2, g0, be0, m0, v0, g1, be1, m1, v1,
           g2, be2, m2, v2, lw0, lb0, lw1, lb1):
    xh = jnp.transpose(x, (0, 2, 1, 3))
    o = _conv_stage(xh, w0, b0, g0, be0, m0, v0, pad=2, P=1, R=49)
    o = _conv_stage(o, w1, b1, g1, be1, m1, v1, pad=1, P=1, R=48)
    o = _conv_stage(o, w2, b2, g2, be2, m2, v2, pad=1, P=1, R=24)
    out = _mlp(o, lw0, lb0, lw1, lb1)
    n = x.shape[0]
    return out.reshape(n, 10, 8)


# speedup vs baseline: 4.1905x; 4.1905x over previous
import functools

import jax
import jax.numpy as jnp
from jax import lax
from jax.experimental import pallas as pl
from jax.experimental.pallas import tpu as pltpu

_BN_EPS = 1e-5


def _pack_conv_w(w_oihw, cinp):
    cout, cin, k, _ = w_oihw.shape
    kp = k + 1
    w = jnp.pad(w_oihw, ((0, 0), (0, cinp - cin), (0, 0), (0, 0)))
    wt = jnp.transpose(w, (2, 3, 0, 1))
    big = jnp.zeros((4, kp, kp, cout, cinp), jnp.float32)
    for dh in range(2):
        for dw in range(2):
            big = big.at[2 * dh + dw, dh:dh + k, dw:dw + k].set(wt)
    return jnp.transpose(big, (0, 3, 2, 1, 4)).reshape(4 * cout, kp * kp * cinp)


def _conv_pool_bn_kernel(x_ref, w_ref, bss_ref, o_ref, *, K, Cinp, Cout, R, P, Bp):
    j = pl.program_id(1)
    kp = K + 1
    bias = bss_ref[0:Cout]
    scale = bss_ref[Cout:2 * Cout]
    shift = bss_ref[2 * Cout:3 * Cout]
    w = w_ref[...]
    for r0 in range(0, R, P):
        taps = []
        for t in range(kp):
            q, c = t % 2, t // 2
            cols = [
                x_ref[0, q,
                      pl.ds(pl.multiple_of(2 * (j * R + r0 + p) * Cinp, 2 * Cinp),
                            kp * Cinp),
                      pl.ds(c, Bp)]
                for p in range(P)
            ]
            taps.append(cols[0] if P == 1 else jnp.concatenate(cols, axis=1))
        slab = taps[0] if kp == 1 else jnp.concatenate(taps, axis=0)
        acc = jnp.dot(w, slab.astype(jnp.bfloat16),
                      preferred_element_type=jnp.float32)
        pooled = jnp.maximum(jnp.maximum(acc[0:Cout], acc[Cout:2 * Cout]),
                             jnp.maximum(acc[2 * Cout:3 * Cout], acc[3 * Cout:]))
        y = jnp.maximum(pooled + bias, 0.0) * scale + shift
        for p in range(P):
            o_ref[0, r0 + p] = y[:, p * Bp:(p + 1) * Bp].astype(o_ref.dtype)


def _conv_stage(x_nhcw, w_oihw, b, gamma, beta, mean, var, *, pad, P, R):
    n, h, cin, w = x_nhcw.shape
    cout, cin_w, k, _ = w_oihw.shape
    cinp = ((cin + 7) // 8) * 8
    hp, wp = h + 2 * pad, w + 2 * pad
    a, bp = (hp - k + 1) // 2, (wp - k + 1) // 2
    kp, wq = k + 1, wp // 2

    xp = jnp.pad(x_nhcw, ((0, 0), (pad, pad), (0, cinp - cin), (pad, pad)))
    xq = xp.reshape(n, hp, cinp, wq, 2).transpose(0, 4, 1, 2, 3)
    xq = xq.reshape(n, 2, hp * cinp, wq)

    wpk = _pack_conv_w(w_oihw, cinp).astype(jnp.bfloat16)
    scale = gamma * lax.rsqrt(var + _BN_EPS)
    shift = beta - mean * scale
    bss = jnp.concatenate([b, scale, shift]).astype(jnp.float32).reshape(3 * cout, 1)

    kern = functools.partial(_conv_pool_bn_kernel, K=k, Cinp=cinp, Cout=cout,
                             R=R, P=P, Bp=bp)
    return pl.pallas_call(
        kern,
        out_shape=jax.ShapeDtypeStruct((n, a, cout, bp), jnp.float32),
        grid_spec=pltpu.PrefetchScalarGridSpec(
            num_scalar_prefetch=0,
            grid=(n, a // R),
            in_specs=[
                pl.BlockSpec((1, 2, hp * cinp, wq), lambda ni, ji: (ni, 0, 0, 0)),
                pl.BlockSpec((4 * cout, kp * kp * cinp), lambda ni, ji: (0, 0)),
                pl.BlockSpec((3 * cout, 1), lambda ni, ji: (0, 0)),
            ],
            out_specs=pl.BlockSpec((1, R, cout, bp), lambda ni, ji: (ni, ji, 0, 0)),
        ),
        compiler_params=pltpu.CompilerParams(
            dimension_semantics=("parallel", "parallel")),
    )(xq, wpk, bss)


def _mlp_kernel(x4_ref, w4_ref, b0_ref, w1_ref, b1_ref, o_ref, *, N, M0):
    y = jnp.dot(x4_ref[...], w4_ref[...], preferred_element_type=jnp.float32)
    h = (y[0:N, 0:M0] + y[N:2 * N, M0:2 * M0] + y[2 * N:3 * N, 2 * M0:3 * M0]
         + y[3 * N:4 * N, 3 * M0:4 * M0] + b0_ref[...])
    h = jnp.maximum(h, 0.0).astype(jnp.bfloat16)
    o_ref[...] = jnp.dot(h, w1_ref[...],
                         preferred_element_type=jnp.float32) + b1_ref[...]


def _mlp(o_nhcw, lw0, lb0, lw1, lb1):
    n, hh, cc, ww = o_nhcw.shape
    kdim, m0 = lw0.shape
    m1 = lw1.shape[1]
    kc = kdim // 4
    lw0p = lw0.reshape(cc, hh, ww, m0).transpose(1, 0, 2, 3).reshape(kdim, m0)
    x = o_nhcw.reshape(n, kdim)
    x4 = jnp.concatenate([x[:, i * kc:(i + 1) * kc] for i in range(4)],
                         axis=0).astype(jnp.bfloat16)
    w4 = jnp.concatenate([lw0p[i * kc:(i + 1) * kc] for i in range(4)],
                         axis=1).astype(jnp.bfloat16)
    return pl.pallas_call(
        functools.partial(_mlp_kernel, N=n, M0=m0),
        out_shape=jax.ShapeDtypeStruct((n, m1), jnp.float32),
        grid_spec=pltpu.PrefetchScalarGridSpec(
            num_scalar_prefetch=0,
            grid=(1,),
            in_specs=[
                pl.BlockSpec((4 * n, kc), lambda i: (0, 0)),
                pl.BlockSpec((kc, 4 * m0), lambda i: (0, 0)),
                pl.BlockSpec((1, m0), lambda i: (0, 0)),
                pl.BlockSpec((m0, m1), lambda i: (0, 0)),
                pl.BlockSpec((1, m1), lambda i: (0, 0)),
            ],
            out_specs=pl.BlockSpec((n, m1), lambda i: (0, 0)),
        ),
        compiler_params=pltpu.CompilerParams(
            dimension_semantics=("arbitrary",),
            vmem_limit_bytes=64 * 1024 * 1024),
    )(x4, w4, lb0.astype(jnp.float32).reshape(1, m0),
      lw1.astype(jnp.bfloat16), lb1.astype(jnp.float32).reshape(1, m1))


def kernel(x, w0, b0, w1, b1, w2, b2, g0, be0, m0, v0, g1, be1, m1, v1,
           g2, be2, m2, v2, lw0, lb0, lw1, lb1):
    xh = jnp.transpose(x, (0, 2, 1, 3))
    o = _conv_stage(xh, w0, b0, g0, be0, m0, v0, pad=2, P=1, R=49)
    o = _conv_stage(o, w1, b1, g1, be1, m1, v1, pad=1, P=1, R=48)
    o = _conv_stage(o, w2, b2, g2, be2, m2, v2, pad=1, P=1, R=24)
    out = _mlp(o, lw0, lb0, lw1, lb1)
    n = x.shape[0]
    return out.reshape(n, 10, 8)


# speedup vs baseline: 9.1865x; 2.1922x over previous
import functools

import jax
import jax.numpy as jnp
from jax import lax
from jax.experimental import pallas as pl
from jax.experimental.pallas import tpu as pltpu

_BN_EPS = 1e-5


def _pack_conv_w(w_oihw, cinp):
    cout, cin, k, _ = w_oihw.shape
    kp = k + 1
    w = jnp.pad(w_oihw, ((0, 0), (0, cinp - cin), (0, 0), (0, 0)))
    wt = jnp.transpose(w, (2, 3, 0, 1))
    big = jnp.zeros((4, kp, kp, cout, cinp), jnp.float32)
    for dh in range(2):
        for dw in range(2):
            big = big.at[2 * dh + dw, dh:dh + k, dw:dw + k].set(wt)
    return jnp.transpose(big, (0, 3, 2, 1, 4)).reshape(4 * cout, kp * kp * cinp)


def _conv_pool_bn_kernel(x_ref, sel_ref, w_ref, bss_ref, o_ref, scr_ref, *,
                         K, Cinp, Cout, R, Wq, Bp):
    j = pl.program_id(1)
    kp = K + 1
    bias = bss_ref[0:Cout]
    scale = bss_ref[Cout:2 * Cout]
    shift = bss_ref[2 * Cout:3 * Cout]

    @pl.when(j == 0)
    def _deinterleave():
        scr_ref[...] = jnp.dot(
            x_ref[0].astype(jnp.bfloat16), sel_ref[...],
            preferred_element_type=jnp.float32).astype(jnp.bfloat16)

    w = w_ref[...]
    for r in range(R):
        start = pl.multiple_of(2 * (j * R + r) * Cinp, 2 * Cinp)
        slab = jnp.concatenate(
            [scr_ref[pl.ds(start, kp * Cinp),
                     pl.ds((t % 2) * Wq + t // 2, Bp)]
             for t in range(kp)], axis=0)
        acc = jnp.dot(w, slab, preferred_element_type=jnp.float32)
        pooled = jnp.maximum(jnp.maximum(acc[0:Cout], acc[Cout:2 * Cout]),
                             jnp.maximum(acc[2 * Cout:3 * Cout], acc[3 * Cout:]))
        y = jnp.maximum(pooled + bias, 0.0) * scale + shift
        o_ref[0, r] = y.astype(o_ref.dtype)


def _conv_stage(x_nhcw, w_oihw, b, gamma, beta, mean, var, *, pad, R):
    n, h, cin, w = x_nhcw.shape
    cout, cin_w, k, _ = w_oihw.shape
    cinp = ((cin + 7) // 8) * 8
    hp, wp = h + 2 * pad, w + 2 * pad
    a, bp = (hp - k + 1) // 2, (wp - k + 1) // 2
    kp, wq = k + 1, wp // 2

    xp = jnp.pad(x_nhcw, ((0, 0), (pad, pad), (0, cinp - cin), (pad, pad)))
    xq = xp.reshape(n, hp * cinp, wp)

    u = jnp.arange(wq)
    sel = jnp.zeros((wp, 2 * wq), jnp.bfloat16)
    sel = sel.at[2 * u, u].set(1).at[2 * u + 1, wq + u].set(1)

    wpk = _pack_conv_w(w_oihw, cinp).astype(jnp.bfloat16)
    scale = gamma * lax.rsqrt(var + _BN_EPS)
    shift = beta - mean * scale
    bss = jnp.concatenate([b, scale, shift]).astype(jnp.float32).reshape(3 * cout, 1)

    kern = functools.partial(_conv_pool_bn_kernel, K=k, Cinp=cinp, Cout=cout,
                             R=R, Wq=wq, Bp=bp)
    return pl.pallas_call(
        kern,
        out_shape=jax.ShapeDtypeStruct((n, a, cout, bp), jnp.float32),
        grid_spec=pltpu.PrefetchScalarGridSpec(
            num_scalar_prefetch=0,
            grid=(n, a // R),
            in_specs=[
                pl.BlockSpec((1, hp * cinp, wp), lambda ni, ji: (ni, 0, 0)),
                pl.BlockSpec((wp, 2 * wq), lambda ni, ji: (0, 0)),
                pl.BlockSpec((4 * cout, kp * kp * cinp), lambda ni, ji: (0, 0)),
                pl.BlockSpec((3 * cout, 1), lambda ni, ji: (0, 0)),
            ],
            out_specs=pl.BlockSpec((1, R, cout, bp), lambda ni, ji: (ni, ji, 0, 0)),
            scratch_shapes=[pltpu.VMEM((hp * cinp, 2 * wq), jnp.bfloat16)],
        ),
        compiler_params=pltpu.CompilerParams(
            dimension_semantics=("parallel", "arbitrary")),
    )(xq, sel, wpk, bss)


def _mlp_kernel(x4_ref, w4_ref, b0_ref, w1_ref, b1_ref, o_ref, *, N, M0):
    y = jnp.dot(x4_ref[...], w4_ref[...], preferred_element_type=jnp.float32)
    h = (y[0:N, 0:M0] + y[N:2 * N, M0:2 * M0] + y[2 * N:3 * N, 2 * M0:3 * M0]
         + y[3 * N:4 * N, 3 * M0:4 * M0] + b0_ref[...])
    h = jnp.maximum(h, 0.0).astype(jnp.bfloat16)
    o_ref[...] = jnp.dot(h, w1_ref[...],
                         preferred_element_type=jnp.float32) + b1_ref[...]


def _mlp(o_nhcw, lw0, lb0, lw1, lb1):
    n, hh, cc, ww = o_nhcw.shape
    kdim, m0 = lw0.shape
    m1 = lw1.shape[1]
    kc = kdim // 4
    lw0p = lw0.reshape(cc, hh, ww, m0).transpose(1, 0, 2, 3).reshape(kdim, m0)
    x = o_nhcw.reshape(n, kdim)
    x4 = jnp.concatenate([x[:, i * kc:(i + 1) * kc] for i in range(4)],
                         axis=0).astype(jnp.bfloat16)
    w4 = jnp.concatenate([lw0p[i * kc:(i + 1) * kc] for i in range(4)],
                         axis=1).astype(jnp.bfloat16)
    return pl.pallas_call(
        functools.partial(_mlp_kernel, N=n, M0=m0),
        out_shape=jax.ShapeDtypeStruct((n, m1), jnp.float32),
        grid_spec=pltpu.PrefetchScalarGridSpec(
            num_scalar_prefetch=0,
            grid=(1,),
            in_specs=[
                pl.BlockSpec((4 * n, kc), lambda i: (0, 0)),
                pl.BlockSpec((kc, 4 * m0), lambda i: (0, 0)),
                pl.BlockSpec((1, m0), lambda i: (0, 0)),
                pl.BlockSpec((m0, m1), lambda i: (0, 0)),
                pl.BlockSpec((1, m1), lambda i: (0, 0)),
            ],
            out_specs=pl.BlockSpec((n, m1), lambda i: (0, 0)),
        ),
        compiler_params=pltpu.CompilerParams(
            dimension_semantics=("arbitrary",),
            vmem_limit_bytes=64 * 1024 * 1024),
    )(x4, w4, lb0.astype(jnp.float32).reshape(1, m0),
      lw1.astype(jnp.bfloat16), lb1.astype(jnp.float32).reshape(1, m1))


def kernel(x, w0, b0, w1, b1, w2, b2, g0, be0, m0, v0, g1, be1, m1, v1,
           g2, be2, m2, v2, lw0, lb0, lw1, lb1):
    xh = jnp.transpose(x, (0, 2, 1, 3))
    o = _conv_stage(xh, w0, b0, g0, be0, m0, v0, pad=2, R=49)
    o = _conv_stage(o, w1, b1, g1, be1, m1, v1, pad=1, R=48)
    o = _conv_stage(o, w2, b2, g2, be2, m2, v2, pad=1, R=24)
    out = _mlp(o, lw0, lb0, lw1, lb1)
    n = x.shape[0]
    return out.reshape(n, 10, 8)


# speedup vs baseline: 9.8353x; 1.0706x over previous
import functools

import jax
import jax.numpy as jnp
from jax import lax
from jax.experimental import pallas as pl
from jax.experimental.pallas import tpu as pltpu

_BN_EPS = 1e-5


def _pack_conv_w(w_oihw, cinp):
    cout, cin, k, _ = w_oihw.shape
    kp = k + 1
    w = jnp.pad(w_oihw, ((0, 0), (0, cinp - cin), (0, 0), (0, 0)))
    wt = jnp.transpose(w, (2, 3, 0, 1))
    big = jnp.zeros((4, kp, kp, cout, cinp), jnp.float32)
    for dh in range(2):
        for dw in range(2):
            big = big.at[2 * dh + dw, dh:dh + k, dw:dw + k].set(wt)
    return jnp.transpose(big, (0, 3, 2, 1, 4)).reshape(4 * cout, kp * kp * cinp)


def _conv_pool_bn_kernel(x_ref, sel_ref, w_ref, bss_ref, o_ref, scr_ref, *,
                         K, Cinp, Cout, R, Wq, Bp):
    j = pl.program_id(1)
    kp = K + 1
    bias = bss_ref[0:Cout]
    scale = bss_ref[Cout:2 * Cout]
    shift = bss_ref[2 * Cout:3 * Cout]

    @pl.when(j == 0)
    def _deinterleave():
        scr_ref[...] = jnp.dot(
            x_ref[0].astype(jnp.bfloat16), sel_ref[...],
            preferred_element_type=jnp.float32).astype(jnp.bfloat16)

    w = w_ref[...]
    for r in range(R):
        start = pl.multiple_of(2 * (j * R + r) * Cinp, 2 * Cinp)
        slab = jnp.concatenate(
            [scr_ref[pl.ds(start, kp * Cinp),
                     pl.ds((t % 2) * Wq + t // 2, Bp)]
             for t in range(kp)], axis=0)
        acc = jnp.dot(w, slab, preferred_element_type=jnp.float32)
        pooled = jnp.maximum(jnp.maximum(acc[0:Cout], acc[Cout:2 * Cout]),
                             jnp.maximum(acc[2 * Cout:3 * Cout], acc[3 * Cout:]))
        y = jnp.maximum(pooled + bias, 0.0) * scale + shift
        o_ref[0, r] = y.astype(o_ref.dtype)


def _conv_stage(x_nhcw, w_oihw, b, gamma, beta, mean, var, *, pad, R):
    n, h, cin, w = x_nhcw.shape
    cout, cin_w, k, _ = w_oihw.shape
    cinp = ((cin + 7) // 8) * 8
    hp, wp = h + 2 * pad, w + 2 * pad
    a, bp = (hp - k + 1) // 2, (wp - k + 1) // 2
    kp, wq = k + 1, wp // 2

    xp = jnp.pad(x_nhcw, ((0, 0), (pad, pad), (0, cinp - cin), (pad, pad)))
    xq = xp.reshape(n, hp * cinp, wp)

    u = jnp.arange(wq)
    sel = jnp.zeros((wp, 2 * wq), jnp.bfloat16)
    sel = sel.at[2 * u, u].set(1).at[2 * u + 1, wq + u].set(1)

    wpk = _pack_conv_w(w_oihw, cinp).astype(jnp.bfloat16)
    scale = gamma * lax.rsqrt(var + _BN_EPS)
    shift = beta - mean * scale
    bss = jnp.concatenate([b, scale, shift]).astype(jnp.float32).reshape(3 * cout, 1)

    kern = functools.partial(_conv_pool_bn_kernel, K=k, Cinp=cinp, Cout=cout,
                             R=R, Wq=wq, Bp=bp)
    return pl.pallas_call(
        kern,
        out_shape=jax.ShapeDtypeStruct((n, a, cout, bp), jnp.bfloat16),
        grid_spec=pltpu.PrefetchScalarGridSpec(
            num_scalar_prefetch=0,
            grid=(n, a // R),
            in_specs=[
                pl.BlockSpec((1, hp * cinp, wp), lambda ni, ji: (ni, 0, 0)),
                pl.BlockSpec((wp, 2 * wq), lambda ni, ji: (0, 0)),
                pl.BlockSpec((4 * cout, kp * kp * cinp), lambda ni, ji: (0, 0)),
                pl.BlockSpec((3 * cout, 1), lambda ni, ji: (0, 0)),
            ],
            out_specs=pl.BlockSpec((1, R, cout, bp), lambda ni, ji: (ni, ji, 0, 0)),
            scratch_shapes=[pltpu.VMEM((hp * cinp, 2 * wq), jnp.bfloat16)],
        ),
        compiler_params=pltpu.CompilerParams(
            dimension_semantics=("parallel", "arbitrary")),
    )(xq, sel, wpk, bss)


def _mlp_kernel(x4_ref, w4_ref, b0_ref, w1_ref, b1_ref, o_ref, *, N, M0):
    y = jnp.dot(x4_ref[...], w4_ref[...], preferred_element_type=jnp.float32)
    h = (y[0:N, 0:M0] + y[N:2 * N, M0:2 * M0] + y[2 * N:3 * N, 2 * M0:3 * M0]
         + y[3 * N:4 * N, 3 * M0:4 * M0] + b0_ref[...])
    h = jnp.maximum(h, 0.0).astype(jnp.bfloat16)
    o_ref[...] = jnp.dot(h, w1_ref[...],
                         preferred_element_type=jnp.float32) + b1_ref[...]


def _mlp(o_nhcw, lw0, lb0, lw1, lb1):
    n, hh, cc, ww = o_nhcw.shape
    kdim, m0 = lw0.shape
    m1 = lw1.shape[1]
    kc = kdim // 4
    lw0p = lw0.reshape(cc, hh, ww, m0).transpose(1, 0, 2, 3).reshape(kdim, m0)
    x = o_nhcw.reshape(n, kdim)
    x4 = jnp.concatenate([x[:, i * kc:(i + 1) * kc] for i in range(4)],
                         axis=0).astype(jnp.bfloat16)
    w4 = jnp.concatenate([lw0p[i * kc:(i + 1) * kc] for i in range(4)],
                         axis=1).astype(jnp.bfloat16)
    return pl.pallas_call(
        functools.partial(_mlp_kernel, N=n, M0=m0),
        out_shape=jax.ShapeDtypeStruct((n, m1), jnp.float32),
        grid_spec=pltpu.PrefetchScalarGridSpec(
            num_scalar_prefetch=0,
            grid=(1,),
            in_specs=[
                pl.BlockSpec((4 * n, kc), lambda i: (0, 0)),
                pl.BlockSpec((kc, 4 * m0), lambda i: (0, 0)),
                pl.BlockSpec((1, m0), lambda i: (0, 0)),
                pl.BlockSpec((m0, m1), lambda i: (0, 0)),
                pl.BlockSpec((1, m1), lambda i: (0, 0)),
            ],
            out_specs=pl.BlockSpec((n, m1), lambda i: (0, 0)),
        ),
        compiler_params=pltpu.CompilerParams(
            dimension_semantics=("arbitrary",),
            vmem_limit_bytes=64 * 1024 * 1024),
    )(x4, w4, lb0.astype(jnp.float32).reshape(1, m0),
      lw1.astype(jnp.bfloat16), lb1.astype(jnp.float32).reshape(1, m1))


def kernel(x, w0, b0, w1, b1, w2, b2, g0, be0, m0, v0, g1, be1, m1, v1,
           g2, be2, m2, v2, lw0, lb0, lw1, lb1):
    xh = jnp.transpose(x, (0, 2, 1, 3))
    o = _conv_stage(xh, w0, b0, g0, be0, m0, v0, pad=2, R=49)
    o = _conv_stage(o, w1, b1, g1, be1, m1, v1, pad=1, R=48)
    o = _conv_stage(o, w2, b2, g2, be2, m2, v2, pad=1, R=24)
    out = _mlp(o, lw0, lb0, lw1, lb1)
    n = x.shape[0]
    return out.reshape(n, 10, 8)


# speedup vs baseline: 10.3008x; 1.0473x over previous
import functools

import jax
import jax.numpy as jnp
from jax import lax
from jax.experimental import pallas as pl
from jax.experimental.pallas import tpu as pltpu

_BN_EPS = 1e-5


def _pack_conv_w(w_oihw, cinp):
    cout, cin, k, _ = w_oihw.shape
    kp = k + 1
    w = jnp.pad(w_oihw, ((0, 0), (0, cinp - cin), (0, 0), (0, 0)))
    wt = jnp.transpose(w, (2, 3, 0, 1))
    big = jnp.zeros((4, kp, kp, cout, cinp), jnp.float32)
    for dh in range(2):
        for dw in range(2):
            big = big.at[2 * dh + dw, dh:dh + k, dw:dw + k].set(wt)
    return jnp.transpose(big, (0, 3, 2, 1, 4)).reshape(4 * cout, kp * kp * cinp)


def _conv_pool_bn_kernel(x_ref, sel_ref, w_ref, bss_ref, o_ref, scr_ref, *,
                         K, Cinp, Cout, R, Wq, Bp):
    j = pl.program_id(1)
    kp = K + 1
    bias = bss_ref[0:Cout]
    scale = bss_ref[Cout:2 * Cout]
    shift = bss_ref[2 * Cout:3 * Cout]

    @pl.when(j == 0)
    def _deinterleave():
        scr_ref[...] = jnp.dot(
            x_ref[0].astype(jnp.bfloat16), sel_ref[...],
            preferred_element_type=jnp.float32).astype(jnp.bfloat16)

    w = w_ref[...]
    for r in range(R):
        start = pl.multiple_of(2 * (j * R + r) * Cinp, 2 * Cinp)
        slab = jnp.concatenate(
            [scr_ref[pl.ds(start, kp * Cinp),
                     pl.ds((t % 2) * Wq + t // 2, Bp)]
             for t in range(kp)], axis=0)
        acc = jnp.dot(w, slab, preferred_element_type=jnp.float32)
        pooled = jnp.maximum(jnp.maximum(acc[0:Cout], acc[Cout:2 * Cout]),
                             jnp.maximum(acc[2 * Cout:3 * Cout], acc[3 * Cout:]))
        y = jnp.maximum(pooled + bias, 0.0) * scale + shift
        o_ref[0, r] = y.astype(o_ref.dtype)


def _conv_stage(x_nhcw, w_oihw, b, gamma, beta, mean, var, *, pad, R):
    n, h, cin, w = x_nhcw.shape
    cout, cin_w, k, _ = w_oihw.shape
    cinp = ((cin + 7) // 8) * 8
    hp, wp = h + 2 * pad, w + 2 * pad
    a, bp = (hp - k + 1) // 2, (wp - k + 1) // 2
    kp, wq = k + 1, wp // 2

    xp = jnp.pad(x_nhcw, ((0, 0), (pad, pad), (0, cinp - cin), (pad, pad)))
    xq = xp.reshape(n, hp * cinp, wp)

    u = jnp.arange(wq)
    sel = jnp.zeros((wp, 2 * wq), jnp.bfloat16)
    sel = sel.at[2 * u, u].set(1).at[2 * u + 1, wq + u].set(1)

    wpk = _pack_conv_w(w_oihw, cinp).astype(jnp.bfloat16)
    scale = gamma * lax.rsqrt(var + _BN_EPS)
    shift = beta - mean * scale
    bss = jnp.concatenate([b, scale, shift]).astype(jnp.float32).reshape(3 * cout, 1)

    kern = functools.partial(_conv_pool_bn_kernel, K=k, Cinp=cinp, Cout=cout,
                             R=R, Wq=wq, Bp=bp)
    return pl.pallas_call(
        kern,
        out_shape=jax.ShapeDtypeStruct((n, a, cout, bp), jnp.bfloat16),
        grid_spec=pltpu.PrefetchScalarGridSpec(
            num_scalar_prefetch=0,
            grid=(n, a // R),
            in_specs=[
                pl.BlockSpec((1, hp * cinp, wp), lambda ni, ji: (ni, 0, 0)),
                pl.BlockSpec((wp, 2 * wq), lambda ni, ji: (0, 0)),
                pl.BlockSpec((4 * cout, kp * kp * cinp), lambda ni, ji: (0, 0)),
                pl.BlockSpec((3 * cout, 1), lambda ni, ji: (0, 0)),
            ],
            out_specs=pl.BlockSpec((1, R, cout, bp), lambda ni, ji: (ni, ji, 0, 0)),
            scratch_shapes=[pltpu.VMEM((hp * cinp, 2 * wq), jnp.bfloat16)],
        ),
        compiler_params=pltpu.CompilerParams(
            dimension_semantics=("parallel", "arbitrary")),
    )(xq, sel, wpk, bss)


def _mlp_kernel(x4_ref, w4_ref, b0_ref, w1_ref, b1_ref, o_ref, *, N, M0):
    y = jnp.dot(x4_ref[...], w4_ref[...], preferred_element_type=jnp.float32)
    h = (y[0:N, 0:M0] + y[N:2 * N, M0:2 * M0] + y[2 * N:3 * N, 2 * M0:3 * M0]
         + y[3 * N:4 * N, 3 * M0:4 * M0] + b0_ref[...])
    h = jnp.maximum(h, 0.0).astype(jnp.bfloat16)
    o_ref[...] = jnp.dot(h, w1_ref[...],
                         preferred_element_type=jnp.float32) + b1_ref[...]


def _mlp(o_nhcw, lw0, lb0, lw1, lb1):
    n, hh, cc, ww = o_nhcw.shape
    kdim, m0 = lw0.shape
    m1 = lw1.shape[1]
    kc = kdim // 4
    lw0p = lw0.reshape(cc, hh, ww, m0).transpose(1, 0, 2, 3).reshape(kdim, m0)
    x = o_nhcw.reshape(n, kdim)
    x4 = jnp.concatenate([x[:, i * kc:(i + 1) * kc] for i in range(4)],
                         axis=0).astype(jnp.bfloat16)
    w4 = jnp.concatenate([lw0p[i * kc:(i + 1) * kc] for i in range(4)],
                         axis=1).astype(jnp.bfloat16)
    return pl.pallas_call(
        functools.partial(_mlp_kernel, N=n, M0=m0),
        out_shape=jax.ShapeDtypeStruct((n, m1), jnp.float32),
        grid_spec=pltpu.PrefetchScalarGridSpec(
            num_scalar_prefetch=0,
            grid=(1,),
            in_specs=[
                pl.BlockSpec((4 * n, kc), lambda i: (0, 0)),
                pl.BlockSpec((kc, 4 * m0), lambda i: (0, 0)),
                pl.BlockSpec((1, m0), lambda i: (0, 0)),
                pl.BlockSpec((m0, m1), lambda i: (0, 0)),
                pl.BlockSpec((1, m1), lambda i: (0, 0)),
            ],
            out_specs=pl.BlockSpec((n, m1), lambda i: (0, 0)),
        ),
        compiler_params=pltpu.CompilerParams(
            dimension_semantics=("arbitrary",),
            vmem_limit_bytes=64 * 1024 * 1024),
    )(x4, w4, lb0.astype(jnp.float32).reshape(1, m0),
      lw1.astype(jnp.bfloat16), lb1.astype(jnp.float32).reshape(1, m1))


def kernel(x, w0, b0, w1, b1, w2, b2, g0, be0, m0, v0, g1, be1, m1, v1,
           g2, be2, m2, v2, lw0, lb0, lw1, lb1):
    xh = jnp.transpose(x, (0, 2, 1, 3))
    o = _conv_stage(xh, w0, b0, g0, be0, m0, v0, pad=2, R=98)
    o = _conv_stage(o, w1, b1, g1, be1, m1, v1, pad=1, R=48)
    o = _conv_stage(o, w2, b2, g2, be2, m2, v2, pad=1, R=24)
    out = _mlp(o, lw0, lb0, lw1, lb1)
    n = x.shape[0]
    return out.reshape(n, 10, 8)
